# Initial kernel scaffold; baseline (speedup 1.0000x reference)
import jax
import jax.numpy as jnp
from jax.experimental import pallas as pl


def kernel(x, embeddings):
    raise NotImplementedError("write your pallas kernel here")


# speedup vs baseline: 10.8970x; 10.8970x over previous
import jax
import jax.numpy as jnp
from jax import lax
from jax.experimental import pallas as pl
from jax.experimental.pallas import tpu as pltpu
from jax.experimental.pallas import tpu_sc as plsc

N_FEAT = 2
HASH_SIZE = 2 ** 19
HASH_MASK = HASH_SIZE - 1
N_POINTS = 262144
N_LVL = 8
NC, NS = 2, 16
NW = NC * NS
PTS_PER_W = N_POINTS // NW
P = 128
N_CHUNKS = PTS_PER_W // P
NIDX = 8 * P * N_FEAT
PRIME_Y = jnp.int32(-1640531535)
PRIME_Z = jnp.int32(805459861)


def _sc_body(x_hbm, gs_hbm, emb_hbm, out_hbm,
             xbuf, gsbuf, ibuf, wbuf, dst, outbuf, sem):
    wid = lax.axis_index("s") * NC + lax.axis_index("c")
    base_pt = wid * PTS_PER_W
    pltpu.sync_copy(gs_hbm, gsbuf)
    iota = lax.iota(jnp.int32, 16)
    zero16 = jnp.zeros((16,), jnp.int32)
    one16 = jnp.full((16,), 1, jnp.int32)
    two16 = jnp.full((16,), 2, jnp.int32)

    def chunk_body(ch, carry):
        row0 = base_pt + ch * P
        pltpu.sync_copy(x_hbm.at[pl.ds(row0, P)], xbuf)

        def level_body(lvl, carry2):
            gsv = gsbuf[lvl]
            wbase = zero16 + lvl * (2 * HASH_SIZE)
            for g in range(8):
                pid = iota + (g * 16)
                px = plsc.load_gather(xbuf, [pid, zero16])
                py = plsc.load_gather(xbuf, [pid, one16])
                pz = plsc.load_gather(xbuf, [pid, two16])
                cx = jnp.minimum(jnp.maximum(px, 0.0), 1.0)
                cy = jnp.minimum(jnp.maximum(py, 0.0), 1.0)
                cz = jnp.minimum(jnp.maximum(pz, 0.0), 1.0)
                bx = (cx / gsv).astype(jnp.int32)
                by = (cy / gsv).astype(jnp.int32)
                bz = (cz / gsv).astype(jnp.int32)
                vmx = bx.astype(jnp.float32) * gsv
                vmy = by.astype(jnp.float32) * gsv
                vmz = bz.astype(jnp.float32) * gsv
                wx = (px - vmx) / ((vmx + gsv) - vmx)
                wy = (py - vmy) / ((vmy + gsv) - vmy)
                wz = (pz - vmz) / ((vmz + gsv) - vmz)
                wbuf[0, pl.ds(g * 16, 16)] = wx
                wbuf[1, pl.ds(g * 16, 16)] = wy
                wbuf[2, pl.ds(g * 16, 16)] = wz
                hx0, hx1 = bx, bx + 1
                hy0 = by * PRIME_Y
                hy1 = hy0 + PRIME_Y
                hz0 = bz * PRIME_Z
                hz1 = hz0 + PRIME_Z
                corners = ((hx0, hy0, hz0), (hx0, hy0, hz1),
                           (hx0, hy1, hz0), (hx0, hy1, hz1),
                           (hx1, hy0, hz0), (hx1, hy0, hz1),
                           (hx1, hy1, hz0), (hx1, hy1, hz1))
                for c, (hx, hy, hz) in enumerate(corners):
                    w0 = ((hx ^ hy ^ hz) & HASH_MASK) * 2 + wbase
                    ibuf[pl.ds(c * P + g * 16, 16)] = w0
                    ibuf[pl.ds(8 * P + c * P + g * 16, 16)] = w0 + 1
            pltpu.async_copy(emb_hbm.at[ibuf], dst, sem).wait()
            col0 = zero16 + 2 * lvl
            for g in range(8):
                pid = iota + (g * 16)
                wx = wbuf[0, pl.ds(g * 16, 16)]
                wy = wbuf[1, pl.ds(g * 16, 16)]
                wz = wbuf[2, pl.ds(g * 16, 16)]
                omx, omy, omz = 1.0 - wx, 1.0 - wy, 1.0 - wz
                v = []
                for c in range(8):
                    v.append((plsc.load_gather(dst, [pid + c * P]),
                              plsc.load_gather(dst, [pid + (8 * P + c * P)])))
                for f in range(N_FEAT):
                    c00 = v[0][f] * omx + v[4][f] * wx
                    c01 = v[1][f] * omx + v[5][f] * wx
                    c10 = v[2][f] * omx + v[6][f] * wx
                    c11 = v[3][f] * omx + v[7][f] * wx
                    c0 = c00 * omy + c10 * wy
                    c1 = c01 * omy + c11 * wy
                    cc = c0 * omz + c1 * wz
                    plsc.store_scatter(outbuf, [pid, col0 + f], cc)
            return carry2

        lax.fori_loop(0, N_LVL, level_body, 0)
        pltpu.sync_copy(outbuf, out_hbm.at[pl.ds(row0, P)])
        return carry

    lax.fori_loop(0, N_CHUNKS, chunk_body, 0)


def kernel(x, embeddings):
    b = jnp.exp((jnp.log(512.0) - jnp.log(16.0)) / (16 - 1))
    gs = jnp.stack([(1.0 - 0.0) / jnp.floor(16.0 * b ** i)
                    for i in range(N_LVL)]).astype(jnp.float32)
    gs2 = jnp.tile(gs[:, None], (1, 16))
    emb1d = embeddings.reshape(-1)
    mesh = plsc.VectorSubcoreMesh(core_axis_name="c", subcore_axis_name="s",
                                  num_cores=NC, num_subcores=NS)
    f = pl.kernel(
        _sc_body,
        out_type=jax.ShapeDtypeStruct((N_POINTS, 16), jnp.float32),
        mesh=mesh,
        compiler_params=pltpu.CompilerParams(needs_layout_passes=False,
                                             use_tc_tiling_on_sc=False),
        scratch_types=[
            pltpu.VMEM((P, 3), jnp.float32),
            pltpu.VMEM((N_LVL, 16), jnp.float32),
            pltpu.VMEM((NIDX,), jnp.int32),
            pltpu.VMEM((3, P), jnp.float32),
            pltpu.VMEM((NIDX,), jnp.float32),
            pltpu.VMEM((P, 16), jnp.float32),
            pltpu.SemaphoreType.DMA,
        ],
    )
    return f(x, gs2, emb1d)


# speedup vs baseline: 62.6310x; 5.7476x over previous
import jax
import jax.numpy as jnp
from jax import lax
from jax.experimental import pallas as pl
from jax.experimental.pallas import tpu as pltpu
from jax.experimental.pallas import tpu_sc as plsc

N_FEAT = 2
HASH_SIZE = 2 ** 19
HASH_MASK = HASH_SIZE - 1
N_POINTS = 262144
N_LVL = 8
NC, NS = 2, 16
NW = NC * NS
PTS_PER_W = N_POINTS // NW
P = 128
N_CHUNKS = PTS_PER_W // P
NIDX = 8 * P * N_FEAT
PRIME_Y = jnp.int32(-1640531535)
PRIME_Z = jnp.int32(805459861)


def _sc_body(x_hbm, gs_hbm, emb_hbm, out_hbm,
             xbuf, gsbuf, ibuf, wbuf, dst, outbuf, sem):
    wid = lax.axis_index("s") * NC + lax.axis_index("c")
    base_pt = wid * PTS_PER_W
    pltpu.sync_copy(gs_hbm, gsbuf)
    iota = lax.iota(jnp.int32, 16)
    zero16 = jnp.zeros((16,), jnp.int32)
    one16 = jnp.full((16,), 1, jnp.int32)
    two16 = jnp.full((16,), 2, jnp.int32)

    def chunk_body(ch, carry):
        row0 = base_pt + ch * P
        pltpu.sync_copy(x_hbm.at[pl.ds(row0, P)], xbuf)

        def level_body(lvl, carry2):
            gsv = gsbuf[lvl]
            wbase = zero16 + lvl * (2 * HASH_SIZE)
            for g in range(8):
                pid = iota + (g * 16)
                px = plsc.load_gather(xbuf, [pid, zero16])
                py = plsc.load_gather(xbuf, [pid, one16])
                pz = plsc.load_gather(xbuf, [pid, two16])
                cx = jnp.minimum(jnp.maximum(px, 0.0), 1.0)
                cy = jnp.minimum(jnp.maximum(py, 0.0), 1.0)
                cz = jnp.minimum(jnp.maximum(pz, 0.0), 1.0)
                bx = (cx / gsv).astype(jnp.int32)
                by = (cy / gsv).astype(jnp.int32)
                bz = (cz / gsv).astype(jnp.int32)
                vmx = bx.astype(jnp.float32) * gsv
                vmy = by.astype(jnp.float32) * gsv
                vmz = bz.astype(jnp.float32) * gsv
                wx = (px - vmx) / ((vmx + gsv) - vmx)
                wy = (py - vmy) / ((vmy + gsv) - vmy)
                wz = (pz - vmz) / ((vmz + gsv) - vmz)
                wbuf[0, pl.ds(g * 16, 16)] = wx
                wbuf[1, pl.ds(g * 16, 16)] = wy
                wbuf[2, pl.ds(g * 16, 16)] = wz
                hx0, hx1 = bx, bx + 1
                hy0 = by * PRIME_Y
                hy1 = hy0 + PRIME_Y
                hz0 = bz * PRIME_Z
                hz1 = hz0 + PRIME_Z
                corners = ((hx0, hy0, hz0), (hx0, hy0, hz1),
                           (hx0, hy1, hz0), (hx0, hy1, hz1),
                           (hx1, hy0, hz0), (hx1, hy0, hz1),
                           (hx1, hy1, hz0), (hx1, hy1, hz1))
                for c, (hx, hy, hz) in enumerate(corners):
                    h = (hx ^ hy ^ hz) & HASH_MASK
                    e = h & 127
                    w0 = wbase + ((h - e) << 1) + e
                    ibuf[pl.ds(c * P + g * 16, 16)] = w0
                    ibuf[pl.ds(8 * P + c * P + g * 16, 16)] = w0 + 128
            pltpu.async_copy(emb_hbm.at[ibuf], dst, sem).wait()
            col0 = zero16 + 2 * lvl
            for g in range(8):
                pid = iota + (g * 16)
                wx = wbuf[0, pl.ds(g * 16, 16)]
                wy = wbuf[1, pl.ds(g * 16, 16)]
                wz = wbuf[2, pl.ds(g * 16, 16)]
                omx, omy, omz = 1.0 - wx, 1.0 - wy, 1.0 - wz
                v = []
                for c in range(8):
                    v.append((plsc.load_gather(dst, [pid + c * P]),
                              plsc.load_gather(dst, [pid + (8 * P + c * P)])))
                for f in range(N_FEAT):
                    c00 = v[0][f] * omx + v[4][f] * wx
                    c01 = v[1][f] * omx + v[5][f] * wx
                    c10 = v[2][f] * omx + v[6][f] * wx
                    c11 = v[3][f] * omx + v[7][f] * wx
                    c0 = c00 * omy + c10 * wy
                    c1 = c01 * omy + c11 * wy
                    cc = c0 * omz + c1 * wz
                    plsc.store_scatter(outbuf, [pid, col0 + f], cc)
            return carry2

        lax.fori_loop(0, N_LVL, level_body, 0)
        pltpu.sync_copy(outbuf, out_hbm.at[pl.ds(row0, P)])
        return carry

    lax.fori_loop(0, N_CHUNKS, chunk_body, 0)


def kernel(x, embeddings):
    b = jnp.exp((jnp.log(512.0) - jnp.log(16.0)) / (16 - 1))
    gs = jnp.stack([(1.0 - 0.0) / jnp.floor(16.0 * b ** i)
                    for i in range(N_LVL)]).astype(jnp.float32)
    gs2 = jnp.tile(gs[:, None], (1, 16))
    emb1d = embeddings.reshape(16, HASH_SIZE // 128, 128, N_FEAT) \
                      .transpose(0, 1, 3, 2).reshape(-1)
    mesh = plsc.VectorSubcoreMesh(core_axis_name="c", subcore_axis_name="s",
                                  num_cores=NC, num_subcores=NS)
    f = pl.kernel(
        _sc_body,
        out_type=jax.ShapeDtypeStruct((N_POINTS, 16), jnp.float32),
        mesh=mesh,
        compiler_params=pltpu.CompilerParams(needs_layout_passes=False,
                                             use_tc_tiling_on_sc=False),
        scratch_types=[
            pltpu.VMEM((P, 3), jnp.float32),
            pltpu.VMEM((N_LVL, 16), jnp.float32),
            pltpu.VMEM((NIDX,), jnp.int32),
            pltpu.VMEM((3, P), jnp.float32),
            pltpu.VMEM((NIDX,), jnp.float32),
            pltpu.VMEM((P, 16), jnp.float32),
            pltpu.SemaphoreType.DMA,
        ],
    )
    return f(x, gs2, emb1d)


# speedup vs baseline: 87.4584x; 1.3964x over previous
import jax
import jax.numpy as jnp
from jax import lax
from jax.experimental import pallas as pl
from jax.experimental.pallas import tpu as pltpu
from jax.experimental.pallas import tpu_sc as plsc

N_FEAT = 2
HASH_SIZE = 2 ** 19
HASH_MASK = HASH_SIZE - 1
N_POINTS = 262144
N_LVL = 8
NC, NS = 2, 16
NW = NC * NS
PTS_PER_W = N_POINTS // NW
P = 128
N_CHUNKS = PTS_PER_W // P
N_STEPS = N_CHUNKS * N_LVL
NIDX = 8 * P * N_FEAT
PRIME_Y = jnp.int32(-1640531535)
PRIME_Z = jnp.int32(805459861)


def _hash_and_issue(t, base_pt, xbuf, gsbuf, ibuf, wbuf, dst, sem, emb_hbm,
                    iota, zero16, one16, two16):
    lvl = t & 7
    gsv = gsbuf[lvl]
    wbase = zero16 + lvl * (2 * HASH_SIZE)
    for g in range(8):
        pid = iota + (g * 16)
        px = plsc.load_gather(xbuf, [pid, zero16])
        py = plsc.load_gather(xbuf, [pid, one16])
        pz = plsc.load_gather(xbuf, [pid, two16])
        cx = jnp.minimum(jnp.maximum(px, 0.0), 1.0)
        cy = jnp.minimum(jnp.maximum(py, 0.0), 1.0)
        cz = jnp.minimum(jnp.maximum(pz, 0.0), 1.0)
        bx = (cx / gsv).astype(jnp.int32)
        by = (cy / gsv).astype(jnp.int32)
        bz = (cz / gsv).astype(jnp.int32)
        vmx = bx.astype(jnp.float32) * gsv
        vmy = by.astype(jnp.float32) * gsv
        vmz = bz.astype(jnp.float32) * gsv
        wx = (px - vmx) / ((vmx + gsv) - vmx)
        wy = (py - vmy) / ((vmy + gsv) - vmy)
        wz = (pz - vmz) / ((vmz + gsv) - vmz)
        wbuf[0, pl.ds(g * 16, 16)] = wx
        wbuf[1, pl.ds(g * 16, 16)] = wy
        wbuf[2, pl.ds(g * 16, 16)] = wz
        hx0, hx1 = bx, bx + 1
        hy0 = by * PRIME_Y
        hy1 = hy0 + PRIME_Y
        hz0 = bz * PRIME_Z
        hz1 = hz0 + PRIME_Z
        corners = ((hx0, hy0, hz0), (hx0, hy0, hz1),
                   (hx0, hy1, hz0), (hx0, hy1, hz1),
                   (hx1, hy0, hz0), (hx1, hy0, hz1),
                   (hx1, hy1, hz0), (hx1, hy1, hz1))
        for c, (hx, hy, hz) in enumerate(corners):
            h = (hx ^ hy ^ hz) & HASH_MASK
            e = h & 127
            w0 = wbase + ((h - e) << 1) + e
            ibuf[pl.ds(c * P + g * 16, 16)] = w0
            ibuf[pl.ds(8 * P + c * P + g * 16, 16)] = w0 + 128
    return pltpu.async_copy(emb_hbm.at[ibuf], dst, sem)


def _interp(t, wbuf, dst, outbuf, iota, zero16):
    lvl = t & 7
    col0 = zero16 + 2 * lvl
    for g in range(8):
        pid = iota + (g * 16)
        wx = wbuf[0, pl.ds(g * 16, 16)]
        wy = wbuf[1, pl.ds(g * 16, 16)]
        wz = wbuf[2, pl.ds(g * 16, 16)]
        omx, omy, omz = 1.0 - wx, 1.0 - wy, 1.0 - wz
        v = []
        for c in range(8):
            v.append((plsc.load_gather(dst, [pid + c * P]),
                      plsc.load_gather(dst, [pid + (8 * P + c * P)])))
        for f in range(N_FEAT):
            c00 = v[0][f] * omx + v[4][f] * wx
            c01 = v[1][f] * omx + v[5][f] * wx
            c10 = v[2][f] * omx + v[6][f] * wx
            c11 = v[3][f] * omx + v[7][f] * wx
            c0 = c00 * omy + c10 * wy
            c1 = c01 * omy + c11 * wy
            cc = c0 * omz + c1 * wz
            plsc.store_scatter(outbuf, [col0 + f, pid], cc)


def _sc_body(x_hbm, gs_hbm, emb_hbm, out_hbm,
             xbuf, gsbuf, ibufA, ibufB, wbufA, wbufB, dstA, dstB, outbuf,
             semA, semB):
    wid = lax.axis_index("s") * NC + lax.axis_index("c")
    base_pt = wid * PTS_PER_W
    blk0 = wid * N_CHUNKS
    pltpu.sync_copy(gs_hbm, gsbuf)
    iota = lax.iota(jnp.int32, 16)
    zero16 = jnp.zeros((16,), jnp.int32)
    one16 = jnp.full((16,), 1, jnp.int32)
    two16 = jnp.full((16,), 2, jnp.int32)

    def stage_x(ch):
        pltpu.sync_copy(x_hbm.at[pl.ds(base_pt + ch * P, P)], xbuf)

    def flush_out(t):
        blk = blk0 + (t >> 3)
        pltpu.sync_copy(outbuf.at[pl.ds(0, 8)], out_hbm.at[0, blk])
        pltpu.sync_copy(outbuf.at[pl.ds(8, 8)], out_hbm.at[1, blk])

    args = (iota, zero16, one16, two16)

    stage_x(0)
    _hash_and_issue(0, base_pt, xbuf, gsbuf, ibufA, wbufA, dstA, semA,
                    emb_hbm, *args)

    def step(t, cur, nxt):
        ibuf_n, wbuf_n, dst_n, sem_n = nxt
        _, wbuf_c, dst_c, sem_c = cur

        @pl.when((t < N_STEPS - 1) & (((t + 1) & 7) == 0))
        def _():
            stage_x((t + 1) >> 3)

        @pl.when(t < N_STEPS - 1)
        def _():
            _hash_and_issue(t + 1, base_pt, xbuf, gsbuf, ibuf_n, wbuf_n,
                            dst_n, sem_n, emb_hbm, *args)

        pltpu.make_async_copy(emb_hbm.at[pl.ds(0, NIDX)], dst_c, sem_c).wait()
        _interp(t, wbuf_c, dst_c, outbuf, iota, zero16)

        @pl.when((t & 7) == 7)
        def _():
            flush_out(t)

    A = (ibufA, wbufA, dstA, semA)
    B = (ibufB, wbufB, dstB, semB)

    def pair_body(k, carry):
        step(2 * k, A, B)
        step(2 * k + 1, B, A)
        return carry

    lax.fori_loop(0, N_STEPS // 2, pair_body, 0)


def kernel(x, embeddings):
    b = jnp.exp((jnp.log(512.0) - jnp.log(16.0)) / (16 - 1))
    gs = jnp.stack([(1.0 - 0.0) / jnp.floor(16.0 * b ** i)
                    for i in range(N_LVL)]).astype(jnp.float32)
    gs2 = jnp.tile(gs[:, None], (1, 16))
    emb1d = embeddings.reshape(16, HASH_SIZE // 128, 128, N_FEAT) \
                      .transpose(0, 1, 3, 2).reshape(-1)
    mesh = plsc.VectorSubcoreMesh(core_axis_name="c", subcore_axis_name="s",
                                  num_cores=NC, num_subcores=NS)
    f = pl.kernel(
        _sc_body,
        out_type=jax.ShapeDtypeStruct((2, N_POINTS // P, 8, P), jnp.float32),
        mesh=mesh,
        compiler_params=pltpu.CompilerParams(needs_layout_passes=False,
                                             use_tc_tiling_on_sc=False),
        scratch_types=[
            pltpu.VMEM((P, 3), jnp.float32),
            pltpu.VMEM((N_LVL, 16), jnp.float32),
            pltpu.VMEM((NIDX,), jnp.int32),
            pltpu.VMEM((NIDX,), jnp.int32),
            pltpu.VMEM((3, P), jnp.float32),
            pltpu.VMEM((3, P), jnp.float32),
            pltpu.VMEM((NIDX,), jnp.float32),
            pltpu.VMEM((NIDX,), jnp.float32),
            pltpu.VMEM((16, P), jnp.float32),
            pltpu.SemaphoreType.DMA,
            pltpu.SemaphoreType.DMA,
        ],
    )
    out_p = f(x, gs2, emb1d)
    return out_p.transpose(1, 3, 0, 2).reshape(N_POINTS, 16)


# speedup vs baseline: 112.7922x; 1.2897x over previous
import jax
import jax.numpy as jnp
from jax import lax
from jax.experimental import pallas as pl
from jax.experimental.pallas import tpu as pltpu
from jax.experimental.pallas import tpu_sc as plsc

N_FEAT = 2
HASH_SIZE = 2 ** 19
HASH_MASK = HASH_SIZE - 1
N_POINTS = 262144
N_LVL = 8
N_CT = 3
N_STRM = N_LVL - N_CT
NC, NS = 2, 16
NW = NC * NS
PTS_PER_W = N_POINTS // NW
P = 128
N_CHUNKS = PTS_PER_W // P
N_STEPS = N_CHUNKS * N_STRM
NIDX = 8 * P * N_FEAT
PRIME_Y = jnp.int32(-1640531535)
PRIME_Z = jnp.int32(805459861)

NV = (17, 21, 26)
CT_ENTRIES = tuple(v * v * v for v in NV)
CT_GROUPS = tuple(-(-e // (16 * NS)) for e in CT_ENTRIES)
CT_SLICE = tuple(g * 16 for g in CT_GROUPS)
CT_PAD = tuple(s * NS for s in CT_SLICE)
CT_OFF = (0, CT_PAD[0], CT_PAD[0] + CT_PAD[1])
CT_TOTAL = CT_PAD[0] + CT_PAD[1] + CT_PAD[2]
SLICE_MAX = max(CT_SLICE)


def _corner_hashes(bx, by, bz):
    hx0, hx1 = bx, bx + 1
    hy0 = by * PRIME_Y
    hy1 = hy0 + PRIME_Y
    hz0 = bz * PRIME_Z
    hz1 = hz0 + PRIME_Z
    return ((hx0, hy0, hz0), (hx0, hy0, hz1),
            (hx0, hy1, hz0), (hx0, hy1, hz1),
            (hx1, hy0, hz0), (hx1, hy0, hz1),
            (hx1, hy1, hz0), (hx1, hy1, hz1))


def _voxel(px, py, pz, gsv):
    cx = jnp.minimum(jnp.maximum(px, 0.0), 1.0)
    cy = jnp.minimum(jnp.maximum(py, 0.0), 1.0)
    cz = jnp.minimum(jnp.maximum(pz, 0.0), 1.0)
    bx = (cx / gsv).astype(jnp.int32)
    by = (cy / gsv).astype(jnp.int32)
    bz = (cz / gsv).astype(jnp.int32)
    vmx = bx.astype(jnp.float32) * gsv
    vmy = by.astype(jnp.float32) * gsv
    vmz = bz.astype(jnp.float32) * gsv
    wx = (px - vmx) / ((vmx + gsv) - vmx)
    wy = (py - vmy) / ((vmy + gsv) - vmy)
    wz = (pz - vmz) / ((vmz + gsv) - vmz)
    return bx, by, bz, wx, wy, wz


def _lerp(v, wx, wy, wz, omx, omy, omz, f):
    c00 = v[0][f] * omx + v[4][f] * wx
    c01 = v[1][f] * omx + v[5][f] * wx
    c10 = v[2][f] * omx + v[6][f] * wx
    c11 = v[3][f] * omx + v[7][f] * wx
    c0 = c00 * omy + c10 * wy
    c1 = c01 * omy + c11 * wy
    return c0 * omz + c1 * wz


def _build_compact(sid, emb_hbm, gsbuf, ibslice, ctslice, spct, tilect, sem,
                   iota):
    for lvl in range(N_CT):
        nv = NV[lvl]
        ent = CT_ENTRIES[lvl]
        base_e = sid * CT_SLICE[lvl]
        wlvl = lvl * (2 * HASH_SIZE)

        def g_body(g, carry, nv=nv, ent=ent, base_e=base_e, wlvl=wlvl,
                   lvl=lvl):
            lpid = g * 16 + iota
            cidx = jnp.minimum(base_e + lpid, ent - 1)
            i = cidx // (nv * nv)
            rem = cidx - i * (nv * nv)
            j = rem // nv
            k = rem - j * nv
            h = (i ^ (j * PRIME_Y) ^ (k * PRIME_Z)) & HASH_MASK
            e = h & 127
            w0 = wlvl + ((h - e) << 1) + e
            plsc.store_scatter(ibslice, [lpid * 2], w0)
            plsc.store_scatter(ibslice, [lpid * 2 + 1], w0 + 128)
            return carry

        lax.fori_loop(0, SLICE_MAX // 16, g_body, 0)
        pltpu.async_copy(emb_hbm.at[ibslice], ctslice, sem).wait()
        nw = 2 * CT_SLICE[lvl]
        pltpu.sync_copy(
            ctslice.at[pl.ds(0, nw)],
            spct.at[pl.ds(2 * CT_OFF[lvl] + sid * nw, nw)])
    plsc.subcore_barrier()
    pltpu.sync_copy(spct, tilect)


def _compute_ct_levels(ch, xbuf, gsbuf, tilect, outbuf, iota, zero16, one16,
                       two16):
    for lvl in range(N_CT):
        nv = NV[lvl]
        nv2 = nv * nv
        woff = 2 * CT_OFF[lvl]
        gsv = gsbuf[lvl]
        col0 = zero16 + 2 * lvl
        for g in range(8):
            pid = iota + (g * 16)
            px = plsc.load_gather(xbuf, [pid, zero16])
            py = plsc.load_gather(xbuf, [pid, one16])
            pz = plsc.load_gather(xbuf, [pid, two16])
            bx, by, bz, wx, wy, wz = _voxel(px, py, pz, gsv)
            ax0 = bx * nv2
            ax1 = ax0 + nv2
            ay0 = by * nv
            ay1 = ay0 + nv
            az0, az1 = bz, bz + 1
            cs = ((ax0, ay0, az0), (ax0, ay0, az1),
                  (ax0, ay1, az0), (ax0, ay1, az1),
                  (ax1, ay0, az0), (ax1, ay0, az1),
                  (ax1, ay1, az0), (ax1, ay1, az1))
            v = []
            for (ax, ay, az) in cs:
                w = ((ax + ay + az) << 1) + woff
                v.append((plsc.load_gather(tilect, [w]),
                          plsc.load_gather(tilect, [w + 1])))
            omx, omy, omz = 1.0 - wx, 1.0 - wy, 1.0 - wz
            for f in range(N_FEAT):
                cc = _lerp(v, wx, wy, wz, omx, omy, omz, f)
                plsc.store_scatter(outbuf, [col0 + f, pid], cc)


def _hash_and_issue(lvl, xbuf, gsbuf, ibuf, wbuf, dst, sem, emb_hbm,
                    iota, zero16, one16, two16):
    gsv = gsbuf[lvl]
    wbase = zero16 + lvl * (2 * HASH_SIZE)
    for g in range(8):
        pid = iota + (g * 16)
        px = plsc.load_gather(xbuf, [pid, zero16])
        py = plsc.load_gather(xbuf, [pid, one16])
        pz = plsc.load_gather(xbuf, [pid, two16])
        bx, by, bz, wx, wy, wz = _voxel(px, py, pz, gsv)
        wbuf[0, pl.ds(g * 16, 16)] = wx
        wbuf[1, pl.ds(g * 16, 16)] = wy
        wbuf[2, pl.ds(g * 16, 16)] = wz
        for c, (hx, hy, hz) in enumerate(_corner_hashes(bx, by, bz)):
            h = (hx ^ hy ^ hz) & HASH_MASK
            e = h & 127
            w0 = wbase + ((h - e) << 1) + e
            ibuf[pl.ds(c * P + g * 16, 16)] = w0
            ibuf[pl.ds(8 * P + c * P + g * 16, 16)] = w0 + 128
    pltpu.async_copy(emb_hbm.at[ibuf], dst, sem)


def _interp(lvl, wbuf, dst, outbuf, iota, zero16):
    col0 = zero16 + 2 * lvl
    for g in range(8):
        pid = iota + (g * 16)
        wx = wbuf[0, pl.ds(g * 16, 16)]
        wy = wbuf[1, pl.ds(g * 16, 16)]
        wz = wbuf[2, pl.ds(g * 16, 16)]
        omx, omy, omz = 1.0 - wx, 1.0 - wy, 1.0 - wz
        v = []
        for c in range(8):
            v.append((plsc.load_gather(dst, [pid + c * P]),
                      plsc.load_gather(dst, [pid + (8 * P + c * P)])))
        for f in range(N_FEAT):
            cc = _lerp(v, wx, wy, wz, omx, omy, omz, f)
            plsc.store_scatter(outbuf, [col0 + f, pid], cc)


def _sc_body(x_hbm, gs_hbm, emb_hbm, out_hbm,
             xbuf, gsbuf, ibufA, ibufB, wbufA, wbufB, dstA, dstB, outbuf,
             ibslice, ctslice, tilect, spct, semA, semB):
    cid = lax.axis_index("c")
    sid = lax.axis_index("s")
    wid = sid * NC + cid
    base_pt = wid * PTS_PER_W
    blk0 = wid * N_CHUNKS
    pltpu.sync_copy(gs_hbm, gsbuf)
    iota = lax.iota(jnp.int32, 16)
    zero16 = jnp.zeros((16,), jnp.int32)
    one16 = jnp.full((16,), 1, jnp.int32)
    two16 = jnp.full((16,), 2, jnp.int32)
    args = (iota, zero16, one16, two16)

    _build_compact(sid, emb_hbm, gsbuf, ibslice, ctslice, spct, tilect, semA,
                   iota)

    def stage_x(ch):
        pltpu.sync_copy(x_hbm.at[pl.ds(base_pt + ch * P, P)], xbuf)

    def passA_body(ch, carry):
        stage_x(ch)
        _compute_ct_levels(ch, xbuf, gsbuf, tilect, outbuf, *args)
        blk = blk0 + ch
        pltpu.sync_copy(outbuf.at[pl.ds(0, 6)], out_hbm.at[0, blk, pl.ds(0, 6)])
        return carry

    lax.fori_loop(0, N_CHUNKS, passA_body, 0)

    def flush_out(t):
        blk = blk0 + t // N_STRM
        pltpu.sync_copy(outbuf.at[pl.ds(6, 2)], out_hbm.at[0, blk, pl.ds(6, 2)])
        pltpu.sync_copy(outbuf.at[pl.ds(8, 8)], out_hbm.at[1, blk])

    stage_x(0)
    _hash_and_issue(N_CT, xbuf, gsbuf, ibufA, wbufA, dstA, semA, emb_hbm,
                    *args)

    def step(t, cur, nxt):
        sl = t - (t // N_STRM) * N_STRM
        lvl = N_CT + sl
        ibuf_n, wbuf_n, dst_n, sem_n = nxt
        _, wbuf_c, dst_c, sem_c = cur

        @pl.when((t < N_STEPS - 1) & (sl == N_STRM - 1))
        def _():
            stage_x(t // N_STRM + 1)

        @pl.when(t < N_STEPS - 1)
        def _():
            sl1 = (t + 1) - ((t + 1) // N_STRM) * N_STRM
            _hash_and_issue(N_CT + sl1, xbuf, gsbuf, ibuf_n, wbuf_n,
                            dst_n, sem_n, emb_hbm, *args)

        pltpu.make_async_copy(emb_hbm.at[pl.ds(0, NIDX)], dst_c, sem_c).wait()
        _interp(lvl, wbuf_c, dst_c, outbuf, iota, zero16)

        @pl.when(sl == N_STRM - 1)
        def _():
            flush_out(t)

    A = (ibufA, wbufA, dstA, semA)
    B = (ibufB, wbufB, dstB, semB)

    def pair_body(k, carry):
        step(2 * k, A, B)
        step(2 * k + 1, B, A)
        return carry

    lax.fori_loop(0, N_STEPS // 2, pair_body, 0)


def kernel(x, embeddings):
    b = jnp.exp((jnp.log(512.0) - jnp.log(16.0)) / (16 - 1))
    gs = jnp.stack([(1.0 - 0.0) / jnp.floor(16.0 * b ** i)
                    for i in range(N_LVL)]).astype(jnp.float32)
    gs2 = jnp.tile(gs[:, None], (1, 16))
    emb1d = embeddings.reshape(16, HASH_SIZE // 128, 128, N_FEAT) \
                      .transpose(0, 1, 3, 2).reshape(-1)
    mesh = plsc.VectorSubcoreMesh(core_axis_name="c", subcore_axis_name="s",
                                  num_cores=NC, num_subcores=NS)
    f = pl.kernel(
        _sc_body,
        out_type=jax.ShapeDtypeStruct((2, N_POINTS // P, 8, P), jnp.float32),
        mesh=mesh,
        compiler_params=pltpu.CompilerParams(needs_layout_passes=False,
                                             use_tc_tiling_on_sc=False),
        scratch_types=[
            pltpu.VMEM((P, 3), jnp.float32),
            pltpu.VMEM((N_LVL, 16), jnp.float32),
            pltpu.VMEM((NIDX,), jnp.int32),
            pltpu.VMEM((NIDX,), jnp.int32),
            pltpu.VMEM((3, P), jnp.float32),
            pltpu.VMEM((3, P), jnp.float32),
            pltpu.VMEM((NIDX,), jnp.float32),
            pltpu.VMEM((NIDX,), jnp.float32),
            pltpu.VMEM((16, P), jnp.float32),
            pltpu.VMEM((2 * SLICE_MAX,), jnp.int32),
            pltpu.VMEM((2 * SLICE_MAX,), jnp.float32),
            pltpu.VMEM((2 * CT_TOTAL,), jnp.float32),
            pltpu.VMEM_SHARED((2 * CT_TOTAL,), jnp.float32),
            pltpu.SemaphoreType.DMA,
            pltpu.SemaphoreType.DMA,
        ],
    )
    out_p = f(x, gs2, emb1d)
    return out_p.transpose(1, 3, 0, 2).reshape(N_POINTS, 16)


# speedup vs baseline: 117.0900x; 1.0381x over previous
import jax
import jax.numpy as jnp
from jax import lax
from jax.experimental import pallas as pl
from jax.experimental.pallas import tpu as pltpu
from jax.experimental.pallas import tpu_sc as plsc

N_FEAT = 2
HASH_SIZE = 2 ** 19
HASH_MASK = HASH_SIZE - 1
N_POINTS = 262144
N_LVL = 8
N_CT = 3
N_STRM = N_LVL - N_CT
NC, NS = 2, 16
NW = NC * NS
PTS_PER_W = N_POINTS // NW
P = 128
N_CHUNKS = PTS_PER_W // P
N_STEPS = N_CHUNKS * N_STRM
NIDX = 8 * P * N_FEAT
PRIME_Y = jnp.int32(-1640531535)
PRIME_Z = jnp.int32(805459861)

NV = (17, 21, 26)
CT_ENTRIES = tuple(v * v * v for v in NV)
CT_GROUPS = tuple(-(-e // (16 * NS)) for e in CT_ENTRIES)
CT_SLICE = tuple(g * 16 for g in CT_GROUPS)
CT_PAD = tuple(s * NS for s in CT_SLICE)
CT_OFF = (0, CT_PAD[0], CT_PAD[0] + CT_PAD[1])
CT_TOTAL = CT_PAD[0] + CT_PAD[1] + CT_PAD[2]
SLICE_MAX = max(CT_SLICE)


def _corner_hashes(bx, by, bz):
    hx0, hx1 = bx, bx + 1
    hy0 = by * PRIME_Y
    hy1 = hy0 + PRIME_Y
    hz0 = bz * PRIME_Z
    hz1 = hz0 + PRIME_Z
    return ((hx0, hy0, hz0), (hx0, hy0, hz1),
            (hx0, hy1, hz0), (hx0, hy1, hz1),
            (hx1, hy0, hz0), (hx1, hy0, hz1),
            (hx1, hy1, hz0), (hx1, hy1, hz1))


def _voxel(px, py, pz, gsv):
    cx = jnp.minimum(jnp.maximum(px, 0.0), 1.0)
    cy = jnp.minimum(jnp.maximum(py, 0.0), 1.0)
    cz = jnp.minimum(jnp.maximum(pz, 0.0), 1.0)
    bx = (cx / gsv).astype(jnp.int32)
    by = (cy / gsv).astype(jnp.int32)
    bz = (cz / gsv).astype(jnp.int32)
    vmx = bx.astype(jnp.float32) * gsv
    vmy = by.astype(jnp.float32) * gsv
    vmz = bz.astype(jnp.float32) * gsv
    wx = (px - vmx) / ((vmx + gsv) - vmx)
    wy = (py - vmy) / ((vmy + gsv) - vmy)
    wz = (pz - vmz) / ((vmz + gsv) - vmz)
    return bx, by, bz, wx, wy, wz


def _lerp(v, wx, wy, wz, omx, omy, omz, f):
    c00 = v[0][f] * omx + v[4][f] * wx
    c01 = v[1][f] * omx + v[5][f] * wx
    c10 = v[2][f] * omx + v[6][f] * wx
    c11 = v[3][f] * omx + v[7][f] * wx
    c0 = c00 * omy + c10 * wy
    c1 = c01 * omy + c11 * wy
    return c0 * omz + c1 * wz


def _build_compact(sid, emb_hbm, gsbuf, ibslice, ctslice, spct, tilect, sem,
                   iota):
    for lvl in range(N_CT):
        nv = NV[lvl]
        ent = CT_ENTRIES[lvl]
        base_e = sid * CT_SLICE[lvl]
        wlvl = lvl * (2 * HASH_SIZE)

        def g_body(g, carry, nv=nv, ent=ent, base_e=base_e, wlvl=wlvl,
                   lvl=lvl):
            lpid = g * 16 + iota
            cidx = jnp.minimum(base_e + lpid, ent - 1)
            i = cidx // (nv * nv)
            rem = cidx - i * (nv * nv)
            j = rem // nv
            k = rem - j * nv
            h = (i ^ (j * PRIME_Y) ^ (k * PRIME_Z)) & HASH_MASK
            e = h & 127
            w0 = wlvl + ((h - e) << 1) + e
            plsc.store_scatter(ibslice, [lpid * 2], w0)
            plsc.store_scatter(ibslice, [lpid * 2 + 1], w0 + 128)
            return carry

        lax.fori_loop(0, SLICE_MAX // 16, g_body, 0)
        pltpu.async_copy(emb_hbm.at[ibslice], ctslice, sem).wait()
        nw = 2 * CT_SLICE[lvl]
        pltpu.sync_copy(
            ctslice.at[pl.ds(0, nw)],
            spct.at[pl.ds(2 * CT_OFF[lvl] + sid * nw, nw)])
    plsc.subcore_barrier()
    pltpu.sync_copy(spct, tilect)


def _compute_ct_level(lvl, xbuf, gsbuf, tilect, outbuf, iota, zero16, one16,
                      two16):
    if True:
        nv = NV[lvl]
        nv2 = nv * nv
        woff = 2 * CT_OFF[lvl]
        gsv = gsbuf[lvl]
        col0 = zero16 + 2 * lvl
        for g in range(8):
            pid = iota + (g * 16)
            px = plsc.load_gather(xbuf, [pid, zero16])
            py = plsc.load_gather(xbuf, [pid, one16])
            pz = plsc.load_gather(xbuf, [pid, two16])
            bx, by, bz, wx, wy, wz = _voxel(px, py, pz, gsv)
            ax0 = bx * nv2
            ax1 = ax0 + nv2
            ay0 = by * nv
            ay1 = ay0 + nv
            az0, az1 = bz, bz + 1
            cs = ((ax0, ay0, az0), (ax0, ay0, az1),
                  (ax0, ay1, az0), (ax0, ay1, az1),
                  (ax1, ay0, az0), (ax1, ay0, az1),
                  (ax1, ay1, az0), (ax1, ay1, az1))
            v = []
            for (ax, ay, az) in cs:
                w = ((ax + ay + az) << 1) + woff
                v.append((plsc.load_gather(tilect, [w]),
                          plsc.load_gather(tilect, [w + 1])))
            omx, omy, omz = 1.0 - wx, 1.0 - wy, 1.0 - wz
            for f in range(N_FEAT):
                cc = _lerp(v, wx, wy, wz, omx, omy, omz, f)
                plsc.store_scatter(outbuf, [col0 + f, pid], cc)


def _hash_and_issue(lvl, xbuf, gsbuf, ibuf, wbuf, dst, sem, emb_hbm,
                    iota, zero16, one16, two16):
    gsv = gsbuf[lvl]
    wbase = zero16 + lvl * (2 * HASH_SIZE)
    for g in range(8):
        pid = iota + (g * 16)
        px = plsc.load_gather(xbuf, [pid, zero16])
        py = plsc.load_gather(xbuf, [pid, one16])
        pz = plsc.load_gather(xbuf, [pid, two16])
        bx, by, bz, wx, wy, wz = _voxel(px, py, pz, gsv)
        wbuf[0, pl.ds(g * 16, 16)] = wx
        wbuf[1, pl.ds(g * 16, 16)] = wy
        wbuf[2, pl.ds(g * 16, 16)] = wz
        for c, (hx, hy, hz) in enumerate(_corner_hashes(bx, by, bz)):
            h = (hx ^ hy ^ hz) & HASH_MASK
            e = h & 127
            w0 = wbase + ((h - e) << 1) + e
            ibuf[pl.ds(c * P + g * 16, 16)] = w0
            ibuf[pl.ds(8 * P + c * P + g * 16, 16)] = w0 + 128
    pltpu.async_copy(emb_hbm.at[ibuf], dst, sem)


def _interp(lvl, wbuf, dst, outbuf, iota, zero16):
    col0 = zero16 + 2 * lvl
    for g in range(8):
        pid = iota + (g * 16)
        wx = wbuf[0, pl.ds(g * 16, 16)]
        wy = wbuf[1, pl.ds(g * 16, 16)]
        wz = wbuf[2, pl.ds(g * 16, 16)]
        omx, omy, omz = 1.0 - wx, 1.0 - wy, 1.0 - wz
        v = []
        for c in range(8):
            v.append((plsc.load_gather(dst, [pid + c * P]),
                      plsc.load_gather(dst, [pid + (8 * P + c * P)])))
        for f in range(N_FEAT):
            cc = _lerp(v, wx, wy, wz, omx, omy, omz, f)
            plsc.store_scatter(outbuf, [col0 + f, pid], cc)


def _sc_body(x_hbm, gs_hbm, emb_hbm, out_hbm,
             xbuf, gsbuf, ibufA, ibufB, wbufA, wbufB, dstA, dstB, outbuf,
             ibslice, ctslice, tilect, spct, semA, semB):
    cid = lax.axis_index("c")
    sid = lax.axis_index("s")
    wid = sid * NC + cid
    base_pt = wid * PTS_PER_W
    blk0 = wid * N_CHUNKS
    pltpu.sync_copy(gs_hbm, gsbuf)
    iota = lax.iota(jnp.int32, 16)
    zero16 = jnp.zeros((16,), jnp.int32)
    one16 = jnp.full((16,), 1, jnp.int32)
    two16 = jnp.full((16,), 2, jnp.int32)
    args = (iota, zero16, one16, two16)

    _build_compact(sid, emb_hbm, gsbuf, ibslice, ctslice, spct, tilect, semA,
                   iota)

    def stage_x(ch):
        pltpu.sync_copy(x_hbm.at[pl.ds(base_pt + ch * P, P)], xbuf)

    def flush_out(t):
        blk = blk0 + t // N_STRM
        pltpu.sync_copy(outbuf.at[pl.ds(0, 8)], out_hbm.at[0, blk])
        pltpu.sync_copy(outbuf.at[pl.ds(8, 8)], out_hbm.at[1, blk])

    stage_x(0)
    _hash_and_issue(N_CT, xbuf, gsbuf, ibufA, wbufA, dstA, semA, emb_hbm,
                    *args)

    def step(t, cur, nxt):
        sl = t - (t // N_STRM) * N_STRM
        lvl = N_CT + sl
        ibuf_n, wbuf_n, dst_n, sem_n = nxt
        _, wbuf_c, dst_c, sem_c = cur

        @pl.when((t < N_STEPS - 1) & (sl == N_STRM - 1))
        def _():
            stage_x(t // N_STRM + 1)

        @pl.when(t < N_STEPS - 1)
        def _():
            sl1 = (t + 1) - ((t + 1) // N_STRM) * N_STRM
            _hash_and_issue(N_CT + sl1, xbuf, gsbuf, ibuf_n, wbuf_n,
                            dst_n, sem_n, emb_hbm, *args)

        for ctl in range(N_CT):
            @pl.when(sl == ctl)
            def _(ctl=ctl):
                _compute_ct_level(ctl, xbuf, gsbuf, tilect, outbuf, *args)

        pltpu.make_async_copy(emb_hbm.at[pl.ds(0, NIDX)], dst_c, sem_c).wait()
        _interp(lvl, wbuf_c, dst_c, outbuf, iota, zero16)

        @pl.when(sl == N_STRM - 1)
        def _():
            flush_out(t)

    A = (ibufA, wbufA, dstA, semA)
    B = (ibufB, wbufB, dstB, semB)

    def pair_body(k, carry):
        step(2 * k, A, B)
        step(2 * k + 1, B, A)
        return carry

    lax.fori_loop(0, N_STEPS // 2, pair_body, 0)


def kernel(x, embeddings):
    b = jnp.exp((jnp.log(512.0) - jnp.log(16.0)) / (16 - 1))
    gs = jnp.stack([(1.0 - 0.0) / jnp.floor(16.0 * b ** i)
                    for i in range(N_LVL)]).astype(jnp.float32)
    gs2 = jnp.tile(gs[:, None], (1, 16))
    emb1d = embeddings.reshape(16, HASH_SIZE // 128, 128, N_FEAT) \
                      .transpose(0, 1, 3, 2).reshape(-1)
    mesh = plsc.VectorSubcoreMesh(core_axis_name="c", subcore_axis_name="s",
                                  num_cores=NC, num_subcores=NS)
    f = pl.kernel(
        _sc_body,
        out_type=jax.ShapeDtypeStruct((2, N_POINTS // P, 8, P), jnp.float32),
        mesh=mesh,
        compiler_params=pltpu.CompilerParams(needs_layout_passes=False,
                                             use_tc_tiling_on_sc=False),
        scratch_types=[
            pltpu.VMEM((P, 3), jnp.float32),
            pltpu.VMEM((N_LVL, 16), jnp.float32),
            pltpu.VMEM((NIDX,), jnp.int32),
            pltpu.VMEM((NIDX,), jnp.int32),
            pltpu.VMEM((3, P), jnp.float32),
            pltpu.VMEM((3, P), jnp.float32),
            pltpu.VMEM((NIDX,), jnp.float32),
            pltpu.VMEM((NIDX,), jnp.float32),
            pltpu.VMEM((16, P), jnp.float32),
            pltpu.VMEM((2 * SLICE_MAX,), jnp.int32),
            pltpu.VMEM((2 * SLICE_MAX,), jnp.float32),
            pltpu.VMEM((2 * CT_TOTAL,), jnp.float32),
            pltpu.VMEM_SHARED((2 * CT_TOTAL,), jnp.float32),
            pltpu.SemaphoreType.DMA,
            pltpu.SemaphoreType.DMA,
        ],
    )
    out_p = f(x, gs2, emb1d)
    return out_p.transpose(1, 3, 0, 2).reshape(N_POINTS, 16)
